# Initial kernel scaffold; baseline (speedup 1.0000x reference)
import jax
import jax.numpy as jnp
from jax.experimental import pallas as pl


def kernel(atoms, bonds, bond_atom_1, bond_atom_2, Wq, bq, Wk, bk, Wv, bv, We, Wskip, bskip, A1, ab1, A2, ab2, P1, pb1, P2, pb2, P3, pb3, B1, bb1, B2, bb2):
    raise NotImplementedError("write your pallas kernel here")


# speedup vs baseline: 1.8636x; 1.8636x over previous
import functools

import jax
import jax.numpy as jnp
from jax.experimental import pallas as pl

D = 32
_INV_SQRT_D = 1.0 / (32.0 ** 0.5)


def _pick_block(n, candidates):
    for c in candidates:
        if n % c == 0 and (c % 8 == 0 or c == n):
            return c
    return n


def _proj_body(x_ref, wq_ref, bq_ref, wk_ref, bk_ref, wv_ref, bv_ref,
               ws_ref, bs_ref, q_ref, kv_ref, sk_ref):
    x = x_ref[...]
    q = jnp.dot(x, wq_ref[...], preferred_element_type=jnp.float32) + bq_ref[...]
    k = jnp.dot(x, wk_ref[...], preferred_element_type=jnp.float32) + bk_ref[...]
    v = jnp.dot(x, wv_ref[...], preferred_element_type=jnp.float32) + bv_ref[...]
    s = jnp.dot(x, ws_ref[...], preferred_element_type=jnp.float32) + bs_ref[...]
    q_ref[...] = q
    kv_ref[...] = jnp.concatenate([k, v], axis=1)
    sk_ref[...] = s


def _atom_proj(atoms, WqT, bq, WkT, bk, WvT, bv, WsT, bs):
    n = atoms.shape[0]
    blk = _pick_block(n, (5000, 4000, 2000, 1000, 800, 400, 200, 40, 8, 1))
    grid = (n // blk,)
    wspec = pl.BlockSpec((D, D), lambda i: (0, 0))
    bspec = pl.BlockSpec((D,), lambda i: (0,))
    return pl.pallas_call(
        _proj_body,
        grid=grid,
        in_specs=[pl.BlockSpec((blk, D), lambda i: (i, 0)),
                  wspec, bspec, wspec, bspec, wspec, bspec, wspec, bspec],
        out_specs=[pl.BlockSpec((blk, D), lambda i: (i, 0)),
                   pl.BlockSpec((blk, 2 * D), lambda i: (i, 0)),
                   pl.BlockSpec((blk, D), lambda i: (i, 0))],
        out_shape=[jax.ShapeDtypeStruct((n, D), jnp.float32),
                   jax.ShapeDtypeStruct((n, 2 * D), jnp.float32),
                   jax.ShapeDtypeStruct((n, D), jnp.float32)],
    )(atoms, WqT, bq, WkT, bk, WvT, bv, WsT, bs)


def _edge_att_body(gq_ref, gkv_ref, b_ref, we_ref, cnum_ref, ex_ref):
    e = jnp.dot(b_ref[...], we_ref[...], preferred_element_type=jnp.float32)
    gkv = gkv_ref[...]
    kj = gkv[:, :D] + e
    vj = gkv[:, D:] + e
    lg = jnp.sum(gq_ref[...] * kj, axis=1) * _INV_SQRT_D
    ex = jnp.exp(lg)
    cnum_ref[...] = ex[:, None] * vj
    ex_ref[...] = ex[:, None]


def _edge_att(gq, gkv, bonds, WeT):
    e = bonds.shape[0]
    blk = _pick_block(e, (6400, 3200, 1600, 800, 400, 200, 100, 50, 10, 1))
    grid = (e // blk,)
    return pl.pallas_call(
        _edge_att_body,
        grid=grid,
        in_specs=[pl.BlockSpec((blk, D), lambda i: (i, 0)),
                  pl.BlockSpec((blk, 2 * D), lambda i: (i, 0)),
                  pl.BlockSpec((blk, D), lambda i: (i, 0)),
                  pl.BlockSpec((D, D), lambda i: (0, 0))],
        out_specs=[pl.BlockSpec((blk, D), lambda i: (i, 0)),
                   pl.BlockSpec((blk, 1), lambda i: (i, 0))],
        out_shape=[jax.ShapeDtypeStruct((e, D), jnp.float32),
                   jax.ShapeDtypeStruct((e, 1), jnp.float32)],
    )(gq, gkv, bonds, WeT)


def _atom_upd_body(num_ref, den_ref, atoms_ref, sk_ref,
                   a1_ref, ab1_ref, a2_ref, ab2_ref, out_ref):
    den = jnp.sum(den_ref[...], axis=1)
    agg = num_ref[...] / (den[:, None] + 1e-16)
    atoms2 = atoms_ref[...] + agg + sk_ref[...]
    h = jnp.maximum(
        jnp.dot(atoms2, a1_ref[...], preferred_element_type=jnp.float32)
        + ab1_ref[...], 0.0)
    out_ref[...] = atoms2 + jnp.dot(h, a2_ref[...],
                                    preferred_element_type=jnp.float32) + ab2_ref[...]


def _atom_update(num, den2d, atoms, sk, A1T, ab1, A2T, ab2):
    n = atoms.shape[0]
    p = den2d.shape[1]
    blk = _pick_block(n, (5000, 4000, 2000, 1000, 800, 400, 200, 40, 8, 1))
    grid = (n // blk,)
    return pl.pallas_call(
        _atom_upd_body,
        grid=grid,
        in_specs=[pl.BlockSpec((blk, D), lambda i: (i, 0)),
                  pl.BlockSpec((blk, p), lambda i: (i, 0)),
                  pl.BlockSpec((blk, D), lambda i: (i, 0)),
                  pl.BlockSpec((blk, D), lambda i: (i, 0)),
                  pl.BlockSpec((D, 2 * D), lambda i: (0, 0)),
                  pl.BlockSpec((2 * D,), lambda i: (0,)),
                  pl.BlockSpec((2 * D, D), lambda i: (0, 0)),
                  pl.BlockSpec((D,), lambda i: (0,))],
        out_specs=pl.BlockSpec((blk, D), lambda i: (i, 0)),
        out_shape=jax.ShapeDtypeStruct((n, D), jnp.float32),
    )(num, den2d, atoms, sk, A1T, ab1, A2T, ab2)


def _edge_mlp_body(ga1_ref, ga2_ref, b_ref, p1_ref, pb1_ref, p2_ref, pb2_ref,
                   p3_ref, pb3_ref, b1_ref, bb1_ref, b2_ref, bb2_ref, out_ref):
    bnd = b_ref[...]
    eb = jnp.concatenate([ga1_ref[...], ga2_ref[...], bnd], axis=1)
    h = jnp.maximum(
        jnp.dot(eb, p1_ref[...], preferred_element_type=jnp.float32) + pb1_ref[...], 0.0)
    h = jnp.maximum(
        jnp.dot(h, p2_ref[...], preferred_element_type=jnp.float32) + pb2_ref[...], 0.0)
    bonds2 = bnd + jnp.dot(h, p3_ref[...],
                           preferred_element_type=jnp.float32) + pb3_ref[...]
    h = jnp.maximum(
        jnp.dot(bonds2, b1_ref[...], preferred_element_type=jnp.float32) + bb1_ref[...],
        0.0)
    out_ref[...] = bonds2 + jnp.dot(h, b2_ref[...],
                                    preferred_element_type=jnp.float32) + bb2_ref[...]


def _edge_mlp(ga1, ga2, bonds, P1T, pb1, P2T, pb2, P3T, pb3, B1T, bb1, B2T, bb2):
    e = bonds.shape[0]
    blk = _pick_block(e, (6400, 3200, 1600, 800, 400, 200, 100, 50, 10, 1))
    grid = (e // blk,)
    def w(shape):
        return pl.BlockSpec(shape, lambda i: tuple(0 for _ in shape))
    return pl.pallas_call(
        _edge_mlp_body,
        grid=grid,
        in_specs=[pl.BlockSpec((blk, D), lambda i: (i, 0)),
                  pl.BlockSpec((blk, D), lambda i: (i, 0)),
                  pl.BlockSpec((blk, D), lambda i: (i, 0)),
                  w((3 * D, 4 * D)), w((4 * D,)),
                  w((4 * D, 2 * D)), w((2 * D,)),
                  w((2 * D, D)), w((D,)),
                  w((D, 2 * D)), w((2 * D,)),
                  w((2 * D, D)), w((D,))],
        out_specs=pl.BlockSpec((blk, D), lambda i: (i, 0)),
        out_shape=jax.ShapeDtypeStruct((e, D), jnp.float32),
    )(ga1, ga2, bonds, P1T, pb1, P2T, pb2, P3T, pb3, B1T, bb1, B2T, bb2)


def kernel(atoms, bonds, bond_atom_1, bond_atom_2,
           Wq, bq, Wk, bk, Wv, bv, We, Wskip, bskip,
           A1, ab1, A2, ab2,
           P1, pb1, P2, pb2, P3, pb3,
           B1, bb1, B2, bb2):
    src = bond_atom_1
    dst = bond_atom_2
    n = atoms.shape[0]

    q, kv, sk = _atom_proj(atoms, Wq.T, bq, Wk.T, bk, Wv.T, bv, Wskip.T, bskip)

    gq = jnp.take(q, dst, axis=0)
    gkv = jnp.take(kv, src, axis=0)

    cnum, ex = _edge_att(gq, gkv, bonds, We.T)

    num = jax.ops.segment_sum(cnum, dst, num_segments=n)
    den = jax.ops.segment_sum(ex[:, 0], dst, num_segments=n)

    atoms3 = _atom_update(num, den[:, None], atoms, sk, A1.T, ab1, A2.T, ab2)

    ga1 = jnp.take(atoms3, src, axis=0)
    ga2 = jnp.take(atoms3, dst, axis=0)

    bonds3 = _edge_mlp(ga1, ga2, bonds, P1.T, pb1, P2.T, pb2, P3.T, pb3,
                       B1.T, bb1, B2.T, bb2)
    return (atoms3, bonds3)


# speedup vs baseline: 3.9998x; 2.1462x over previous
import functools

import jax
import jax.numpy as jnp
from jax import lax
from jax.experimental import pallas as pl
from jax.experimental.pallas import tpu as pltpu
from jax.experimental.pallas import tpu_sc as plsc

D = 32
_INV_SQRT_D = 1.0 / (32.0 ** 0.5)


def _pick_block(n, candidates):
    for c in candidates:
        if n % c == 0 and (c % 8 == 0 or c == n):
            return c
    return n


def _sc_gather2(table, i1, i2, chunk=400):
    e = i1.shape[0]
    dt = table.shape[1]
    info = plsc.get_sparse_core_info()
    nw = info.num_cores * info.num_subcores
    per_w = e // nw
    nch = per_w // chunk
    mesh = plsc.VectorSubcoreMesh(core_axis_name="c", subcore_axis_name="s")

    @functools.partial(
        pl.kernel, mesh=mesh,
        out_type=[jax.ShapeDtypeStruct((e, dt), jnp.float32),
                  jax.ShapeDtypeStruct((e, dt), jnp.float32)],
        scratch_types=[pltpu.VMEM((chunk,), jnp.int32),
                       pltpu.VMEM((chunk,), jnp.int32),
                       pltpu.VMEM((chunk, dt), jnp.float32),
                       pltpu.VMEM((chunk, dt), jnp.float32),
                       pltpu.SemaphoreType.DMA,
                       pltpu.SemaphoreType.DMA])
    def gk(t_hbm, i1_hbm, i2_hbm, o1_hbm, o2_hbm,
           i1_v, i2_v, r1_v, r2_v, s1, s2):
        wid = lax.axis_index("s") * info.num_cores + lax.axis_index("c")
        base0 = wid * per_w

        def body(j, carry):
            base = base0 + j * chunk
            pltpu.sync_copy(i1_hbm.at[pl.ds(base, chunk)], i1_v)
            pltpu.sync_copy(i2_hbm.at[pl.ds(base, chunk)], i2_v)
            c1 = pltpu.async_copy(t_hbm.at[i1_v], r1_v, s1)
            c2 = pltpu.async_copy(t_hbm.at[i2_v], r2_v, s2)
            c1.wait()
            c2.wait()
            pltpu.sync_copy(r1_v, o1_hbm.at[pl.ds(base, chunk)])
            pltpu.sync_copy(r2_v, o2_hbm.at[pl.ds(base, chunk)])
            return carry

        lax.fori_loop(0, nch, body, 0)

    return gk(table, i1, i2)


def _proj_body(x_ref, wq_ref, bq_ref, wk_ref, bk_ref, wv_ref, bv_ref,
               ws_ref, bs_ref, t_ref):
    x = x_ref[...]
    q = jnp.dot(x, wq_ref[...], preferred_element_type=jnp.float32) + bq_ref[...]
    k = jnp.dot(x, wk_ref[...], preferred_element_type=jnp.float32) + bk_ref[...]
    v = jnp.dot(x, wv_ref[...], preferred_element_type=jnp.float32) + bv_ref[...]
    s = jnp.dot(x, ws_ref[...], preferred_element_type=jnp.float32) + bs_ref[...]
    t_ref[...] = jnp.concatenate([k, v, q, s], axis=1)


def _atom_proj(atoms, WqT, bq, WkT, bk, WvT, bv, WsT, bs):
    n = atoms.shape[0]
    blk = _pick_block(n, (5000, 4000, 2000, 1000, 800, 400, 200, 40, 8, 1))
    grid = (n // blk,)
    wspec = pl.BlockSpec((D, D), lambda i: (0, 0))
    bspec = pl.BlockSpec((D,), lambda i: (0,))
    return pl.pallas_call(
        _proj_body,
        grid=grid,
        in_specs=[pl.BlockSpec((blk, D), lambda i: (i, 0)),
                  wspec, bspec, wspec, bspec, wspec, bspec, wspec, bspec],
        out_specs=pl.BlockSpec((blk, 4 * D), lambda i: (i, 0)),
        out_shape=jax.ShapeDtypeStruct((n, 4 * D), jnp.float32),
    )(atoms, WqT, bq, WkT, bk, WvT, bv, WsT, bs)


def _edge_att_body(g1_ref, g2_ref, b_ref, we_ref, cnum_ref, ex_ref):
    e = jnp.dot(b_ref[...], we_ref[...], preferred_element_type=jnp.float32)
    g1 = g1_ref[...]
    kj = g1[:, :D] + e
    vj = g1[:, D:2 * D] + e
    lg = jnp.sum(g2_ref[:, 2 * D:3 * D] * kj, axis=1) * _INV_SQRT_D
    ex = jnp.exp(lg)
    cnum_ref[...] = ex[:, None] * vj
    ex_ref[...] = ex[:, None]


def _edge_att(g1, g2, bonds, WeT):
    e = bonds.shape[0]
    blk = _pick_block(e, (6400, 3200, 1600, 800, 400, 200, 100, 50, 10, 1))
    grid = (e // blk,)
    return pl.pallas_call(
        _edge_att_body,
        grid=grid,
        in_specs=[pl.BlockSpec((blk, 4 * D), lambda i: (i, 0)),
                  pl.BlockSpec((blk, 4 * D), lambda i: (i, 0)),
                  pl.BlockSpec((blk, D), lambda i: (i, 0)),
                  pl.BlockSpec((D, D), lambda i: (0, 0))],
        out_specs=[pl.BlockSpec((blk, D), lambda i: (i, 0)),
                   pl.BlockSpec((blk, 1), lambda i: (i, 0))],
        out_shape=[jax.ShapeDtypeStruct((e, D), jnp.float32),
                   jax.ShapeDtypeStruct((e, 1), jnp.float32)],
    )(g1, g2, bonds, WeT)


def _atom_upd_body(num_ref, den_ref, atoms_ref, t_ref,
                   a1_ref, ab1_ref, a2_ref, ab2_ref, out_ref, a3t_ref):
    den = jnp.sum(den_ref[...], axis=1)
    agg = num_ref[...] / (den[:, None] + 1e-16)
    atoms2 = atoms_ref[...] + agg + t_ref[:, 3 * D:]
    h = jnp.maximum(
        jnp.dot(atoms2, a1_ref[...], preferred_element_type=jnp.float32)
        + ab1_ref[...], 0.0)
    a3 = atoms2 + jnp.dot(h, a2_ref[...],
                          preferred_element_type=jnp.float32) + ab2_ref[...]
    out_ref[...] = a3
    a3t_ref[...] = jnp.concatenate(
        [a3, a3, jnp.zeros((a3.shape[0], 2 * D), jnp.float32)], axis=1)


def _atom_update(num, den2d, atoms, table, A1T, ab1, A2T, ab2):
    n = atoms.shape[0]
    p = den2d.shape[1]
    blk = _pick_block(n, (5000, 4000, 2000, 1000, 800, 400, 200, 40, 8, 1))
    grid = (n // blk,)
    return pl.pallas_call(
        _atom_upd_body,
        grid=grid,
        in_specs=[pl.BlockSpec((blk, D), lambda i: (i, 0)),
                  pl.BlockSpec((blk, p), lambda i: (i, 0)),
                  pl.BlockSpec((blk, D), lambda i: (i, 0)),
                  pl.BlockSpec((blk, 4 * D), lambda i: (i, 0)),
                  pl.BlockSpec((D, 2 * D), lambda i: (0, 0)),
                  pl.BlockSpec((2 * D,), lambda i: (0,)),
                  pl.BlockSpec((2 * D, D), lambda i: (0, 0)),
                  pl.BlockSpec((D,), lambda i: (0,))],
        out_specs=[pl.BlockSpec((blk, D), lambda i: (i, 0)),
                   pl.BlockSpec((blk, 4 * D), lambda i: (i, 0))],
        out_shape=[jax.ShapeDtypeStruct((n, D), jnp.float32),
                   jax.ShapeDtypeStruct((n, 4 * D), jnp.float32)],
    )(num, den2d, atoms, table, A1T, ab1, A2T, ab2)


def _edge_mlp_body(ga1_ref, ga2_ref, b_ref, p1_ref, pb1_ref, p2_ref, pb2_ref,
                   p3_ref, pb3_ref, b1_ref, bb1_ref, b2_ref, bb2_ref, out_ref):
    bnd = b_ref[...]
    eb = jnp.concatenate([ga1_ref[:, :D], ga2_ref[:, :D], bnd], axis=1)
    h = jnp.maximum(
        jnp.dot(eb, p1_ref[...], preferred_element_type=jnp.float32) + pb1_ref[...], 0.0)
    h = jnp.maximum(
        jnp.dot(h, p2_ref[...], preferred_element_type=jnp.float32) + pb2_ref[...], 0.0)
    bonds2 = bnd + jnp.dot(h, p3_ref[...],
                           preferred_element_type=jnp.float32) + pb3_ref[...]
    h = jnp.maximum(
        jnp.dot(bonds2, b1_ref[...], preferred_element_type=jnp.float32) + bb1_ref[...],
        0.0)
    out_ref[...] = bonds2 + jnp.dot(h, b2_ref[...],
                                    preferred_element_type=jnp.float32) + bb2_ref[...]


def _edge_mlp(ga1, ga2, bonds, P1T, pb1, P2T, pb2, P3T, pb3, B1T, bb1, B2T, bb2):
    e = bonds.shape[0]
    blk = _pick_block(e, (6400, 3200, 1600, 800, 400, 200, 100, 50, 10, 1))
    grid = (e // blk,)
    def w(shape):
        return pl.BlockSpec(shape, lambda i: tuple(0 for _ in shape))
    return pl.pallas_call(
        _edge_mlp_body,
        grid=grid,
        in_specs=[pl.BlockSpec((blk, 4 * D), lambda i: (i, 0)),
                  pl.BlockSpec((blk, 4 * D), lambda i: (i, 0)),
                  pl.BlockSpec((blk, D), lambda i: (i, 0)),
                  w((3 * D, 4 * D)), w((4 * D,)),
                  w((4 * D, 2 * D)), w((2 * D,)),
                  w((2 * D, D)), w((D,)),
                  w((D, 2 * D)), w((2 * D,)),
                  w((2 * D, D)), w((D,))],
        out_specs=pl.BlockSpec((blk, D), lambda i: (i, 0)),
        out_shape=jax.ShapeDtypeStruct((e, D), jnp.float32),
    )(ga1, ga2, bonds, P1T, pb1, P2T, pb2, P3T, pb3, B1T, bb1, B2T, bb2)


def kernel(atoms, bonds, bond_atom_1, bond_atom_2,
           Wq, bq, Wk, bk, Wv, bv, We, Wskip, bskip,
           A1, ab1, A2, ab2,
           P1, pb1, P2, pb2, P3, pb3,
           B1, bb1, B2, bb2):
    src = bond_atom_1
    dst = bond_atom_2
    n = atoms.shape[0]

    table = _atom_proj(atoms, Wq.T, bq, Wk.T, bk, Wv.T, bv, Wskip.T, bskip)

    g1, g2 = _sc_gather2(table, src, dst)

    cnum, ex = _edge_att(g1, g2, bonds, We.T)

    num = jax.ops.segment_sum(cnum, dst, num_segments=n)
    den = jax.ops.segment_sum(ex[:, 0], dst, num_segments=n)

    atoms3, a3t = _atom_update(num, den[:, None], atoms, table,
                               A1.T, ab1, A2.T, ab2)

    ga1, ga2 = _sc_gather2(a3t, src, dst)

    bonds3 = _edge_mlp(ga1, ga2, bonds, P1.T, pb1, P2.T, pb2, P3.T, pb3,
                       B1.T, bb1, B2.T, bb2)
    return (atoms3, bonds3)


# speedup vs baseline: 4.0027x; 1.0007x over previous
import functools

import jax
import jax.numpy as jnp
from jax import lax
from jax.experimental import pallas as pl
from jax.experimental.pallas import tpu as pltpu
from jax.experimental.pallas import tpu_sc as plsc

D = 32
_INV_SQRT_D = 1.0 / (32.0 ** 0.5)


def _pick_block(n, candidates):
    for c in candidates:
        if n % c == 0 and (c % 8 == 0 or c == n):
            return c
    return n


def _sc_gather2(table, i1, i2, chunk=400):
    e = i1.shape[0]
    dt = table.shape[1]
    info = plsc.get_sparse_core_info()
    nw = info.num_cores * info.num_subcores
    per_w = e // nw
    nch = per_w // chunk
    mesh = plsc.VectorSubcoreMesh(core_axis_name="c", subcore_axis_name="s")

    @functools.partial(
        pl.kernel, mesh=mesh,
        out_type=[jax.ShapeDtypeStruct((e, dt), jnp.float32),
                  jax.ShapeDtypeStruct((e, dt), jnp.float32)],
        scratch_types=[pltpu.VMEM((chunk,), jnp.int32),
                       pltpu.VMEM((chunk,), jnp.int32),
                       pltpu.VMEM((chunk, dt), jnp.float32),
                       pltpu.VMEM((chunk, dt), jnp.float32),
                       pltpu.SemaphoreType.DMA,
                       pltpu.SemaphoreType.DMA])
    def gk(t_hbm, i1_hbm, i2_hbm, o1_hbm, o2_hbm,
           i1_v, i2_v, r1_v, r2_v, s1, s2):
        wid = lax.axis_index("s") * info.num_cores + lax.axis_index("c")
        base0 = wid * per_w

        def body(j, carry):
            base = base0 + j * chunk
            pltpu.sync_copy(i1_hbm.at[pl.ds(base, chunk)], i1_v)
            pltpu.sync_copy(i2_hbm.at[pl.ds(base, chunk)], i2_v)
            c1 = pltpu.async_copy(t_hbm.at[i1_v], r1_v, s1)
            c2 = pltpu.async_copy(t_hbm.at[i2_v], r2_v, s2)
            c1.wait()
            c2.wait()
            pltpu.sync_copy(r1_v, o1_hbm.at[pl.ds(base, chunk)])
            pltpu.sync_copy(r2_v, o2_hbm.at[pl.ds(base, chunk)])
            return carry

        lax.fori_loop(0, nch, body, 0)

    return gk(table, i1, i2)


def _proj_body(x_ref, wq_ref, bq_ref, wk_ref, bk_ref, wv_ref, bv_ref,
               ws_ref, bs_ref, t_ref):
    x = x_ref[...]
    q = jnp.dot(x, wq_ref[...], preferred_element_type=jnp.float32) + bq_ref[...]
    k = jnp.dot(x, wk_ref[...], preferred_element_type=jnp.float32) + bk_ref[...]
    v = jnp.dot(x, wv_ref[...], preferred_element_type=jnp.float32) + bv_ref[...]
    s = jnp.dot(x, ws_ref[...], preferred_element_type=jnp.float32) + bs_ref[...]
    t_ref[...] = jnp.concatenate([k, v, q, s], axis=1)


def _atom_proj(atoms, WqT, bq, WkT, bk, WvT, bv, WsT, bs):
    n = atoms.shape[0]
    blk = _pick_block(n, (5000, 4000, 2000, 1000, 800, 400, 200, 40, 8, 1))
    grid = (n // blk,)
    wspec = pl.BlockSpec((D, D), lambda i: (0, 0))
    bspec = pl.BlockSpec((D,), lambda i: (0,))
    return pl.pallas_call(
        _proj_body,
        grid=grid,
        in_specs=[pl.BlockSpec((blk, D), lambda i: (i, 0)),
                  wspec, bspec, wspec, bspec, wspec, bspec, wspec, bspec],
        out_specs=pl.BlockSpec((blk, 4 * D), lambda i: (i, 0)),
        out_shape=jax.ShapeDtypeStruct((n, 4 * D), jnp.float32),
    )(atoms, WqT, bq, WkT, bk, WvT, bv, WsT, bs)


def _edge_att_body(g1_ref, g2_ref, b_ref, we_ref, c_ref, ex_ref):
    e = jnp.dot(b_ref[...], we_ref[...], preferred_element_type=jnp.float32)
    g1 = g1_ref[...]
    kj = g1[:, :D] + e
    vj = g1[:, D:2 * D] + e
    lg = jnp.sum(g2_ref[:, 2 * D:3 * D] * kj, axis=1) * _INV_SQRT_D
    ex = jnp.exp(lg)
    c_ref[...] = ex[:, None] * vj
    ex_ref[...] = ex[:, None]


def _edge_att(g1, g2, bonds, WeT):
    e = bonds.shape[0]
    blk = _pick_block(e, (6400, 3200, 1600, 800, 400, 200, 100, 50, 10, 1))
    grid = (e // blk,)
    return pl.pallas_call(
        _edge_att_body,
        grid=grid,
        in_specs=[pl.BlockSpec((blk, 4 * D), lambda i: (i, 0)),
                  pl.BlockSpec((blk, 4 * D), lambda i: (i, 0)),
                  pl.BlockSpec((blk, D), lambda i: (i, 0)),
                  pl.BlockSpec((D, D), lambda i: (0, 0))],
        out_specs=[pl.BlockSpec((blk, D), lambda i: (i, 0)),
                   pl.BlockSpec((blk, 1), lambda i: (i, 0))],
        out_shape=[jax.ShapeDtypeStruct((e, D), jnp.float32),
                   jax.ShapeDtypeStruct((e, 1), jnp.float32)],
    )(g1, g2, bonds, WeT)


def _atom_upd_body(num_ref, den_ref, atoms_ref, t_ref,
                   a1_ref, ab1_ref, a2_ref, ab2_ref, out_ref, a3t_ref):
    den = jnp.sum(den_ref[...], axis=1)
    agg = num_ref[...] / (den[:, None] + 1e-16)
    atoms2 = atoms_ref[...] + agg + t_ref[:, 3 * D:]
    h = jnp.maximum(
        jnp.dot(atoms2, a1_ref[...], preferred_element_type=jnp.float32)
        + ab1_ref[...], 0.0)
    a3 = atoms2 + jnp.dot(h, a2_ref[...],
                          preferred_element_type=jnp.float32) + ab2_ref[...]
    out_ref[...] = a3
    a3t_ref[...] = jnp.concatenate(
        [a3, a3, jnp.zeros((a3.shape[0], 2 * D), jnp.float32)], axis=1)


def _atom_update(num, den2d, atoms, table, A1T, ab1, A2T, ab2):
    n = atoms.shape[0]
    p = den2d.shape[1]
    blk = 5000
    grid = (n // blk,)
    return pl.pallas_call(
        _atom_upd_body,
        grid=grid,
        in_specs=[pl.BlockSpec((blk, D), lambda i: (i, 0)),
                  pl.BlockSpec((blk, p), lambda i: (i, 0)),
                  pl.BlockSpec((blk, D), lambda i: (i, 0)),
                  pl.BlockSpec((blk, 4 * D), lambda i: (i, 0)),
                  pl.BlockSpec((D, 2 * D), lambda i: (0, 0)),
                  pl.BlockSpec((2 * D,), lambda i: (0,)),
                  pl.BlockSpec((2 * D, D), lambda i: (0, 0)),
                  pl.BlockSpec((D,), lambda i: (0,))],
        out_specs=[pl.BlockSpec((blk, D), lambda i: (i, 0)),
                   pl.BlockSpec((blk, 4 * D), lambda i: (i, 0))],
        out_shape=[jax.ShapeDtypeStruct((n, D), jnp.float32),
                   jax.ShapeDtypeStruct((n, 4 * D), jnp.float32)],
    )(num, den2d, atoms, table, A1T, ab1, A2T, ab2)


def _edge_mlp_body(ga1_ref, ga2_ref, b_ref, p1_ref, pb1_ref, p2_ref, pb2_ref,
                   p3_ref, pb3_ref, b1_ref, bb1_ref, b2_ref, bb2_ref, out_ref):
    bnd = b_ref[...]
    eb = jnp.concatenate([ga1_ref[:, :D], ga2_ref[:, :D], bnd], axis=1)
    h = jnp.maximum(
        jnp.dot(eb, p1_ref[...], preferred_element_type=jnp.float32) + pb1_ref[...], 0.0)
    h = jnp.maximum(
        jnp.dot(h, p2_ref[...], preferred_element_type=jnp.float32) + pb2_ref[...], 0.0)
    bonds2 = bnd + jnp.dot(h, p3_ref[...],
                           preferred_element_type=jnp.float32) + pb3_ref[...]
    h = jnp.maximum(
        jnp.dot(bonds2, b1_ref[...], preferred_element_type=jnp.float32) + bb1_ref[...],
        0.0)
    out_ref[...] = bonds2 + jnp.dot(h, b2_ref[...],
                                    preferred_element_type=jnp.float32) + bb2_ref[...]


def _edge_mlp(ga1, ga2, bonds, P1T, pb1, P2T, pb2, P3T, pb3, B1T, bb1, B2T, bb2):
    e = bonds.shape[0]
    blk = _pick_block(e, (6400, 3200, 1600, 800, 400, 200, 100, 50, 10, 1))
    grid = (e // blk,)
    def w(shape):
        return pl.BlockSpec(shape, lambda i: tuple(0 for _ in shape))
    return pl.pallas_call(
        _edge_mlp_body,
        grid=grid,
        in_specs=[pl.BlockSpec((blk, 4 * D), lambda i: (i, 0)),
                  pl.BlockSpec((blk, 4 * D), lambda i: (i, 0)),
                  pl.BlockSpec((blk, D), lambda i: (i, 0)),
                  w((3 * D, 4 * D)), w((4 * D,)),
                  w((4 * D, 2 * D)), w((2 * D,)),
                  w((2 * D, D)), w((D,)),
                  w((D, 2 * D)), w((2 * D,)),
                  w((2 * D, D)), w((D,))],
        out_specs=pl.BlockSpec((blk, D), lambda i: (i, 0)),
        out_shape=jax.ShapeDtypeStruct((e, D), jnp.float32),
    )(ga1, ga2, bonds, P1T, pb1, P2T, pb2, P3T, pb3, B1T, bb1, B2T, bb2)


def kernel(atoms, bonds, bond_atom_1, bond_atom_2,
           Wq, bq, Wk, bk, Wv, bv, We, Wskip, bskip,
           A1, ab1, A2, ab2,
           P1, pb1, P2, pb2, P3, pb3,
           B1, bb1, B2, bb2):
    src = bond_atom_1
    dst = bond_atom_2
    n = atoms.shape[0]

    table = _atom_proj(atoms, Wq.T, bq, Wk.T, bk, Wv.T, bv, Wskip.T, bskip)

    g1, g2 = _sc_gather2(table, src, dst)

    cnum, ex2d = _edge_att(g1, g2, bonds, We.T)

    num = jax.ops.segment_sum(cnum, dst, num_segments=n)
    den = jax.ops.segment_sum(ex2d[:, 0], dst, num_segments=n)

    atoms3, a3t = _atom_update(num, den[:, None], atoms, table,
                               A1.T, ab1, A2.T, ab2)

    ga1, ga2 = _sc_gather2(a3t, src, dst)

    bonds3 = _edge_mlp(ga1, ga2, bonds, P1.T, pb1, P2.T, pb2, P3.T, pb3,
                       B1.T, bb1, B2.T, bb2)
    return (atoms3, bonds3)


# speedup vs baseline: 4.1138x; 1.0277x over previous
import functools

import jax
import jax.numpy as jnp
from jax import lax
from jax.experimental import pallas as pl
from jax.experimental.pallas import tpu as pltpu
from jax.experimental.pallas import tpu_sc as plsc

D = 32
_INV_SQRT_D = 1.0 / (32.0 ** 0.5)


def _pick_block(n, candidates):
    for c in candidates:
        if n % c == 0 and (c % 8 == 0 or c == n):
            return c
    return n


def _sc_gather2(table, i1, i2, chunk=400):
    e = i1.shape[0]
    dt = table.shape[1]
    info = plsc.get_sparse_core_info()
    nw = info.num_cores * info.num_subcores
    per_w = e // nw
    nch = per_w // chunk
    mesh = plsc.VectorSubcoreMesh(core_axis_name="c", subcore_axis_name="s")

    @functools.partial(
        pl.kernel, mesh=mesh,
        out_type=[jax.ShapeDtypeStruct((e, dt), jnp.float32),
                  jax.ShapeDtypeStruct((e, dt), jnp.float32)],
        scratch_types=[pltpu.VMEM((chunk,), jnp.int32),
                       pltpu.VMEM((chunk,), jnp.int32),
                       pltpu.VMEM((chunk, dt), jnp.float32),
                       pltpu.VMEM((chunk, dt), jnp.float32),
                       pltpu.SemaphoreType.DMA,
                       pltpu.SemaphoreType.DMA])
    def gk(t_hbm, i1_hbm, i2_hbm, o1_hbm, o2_hbm,
           i1_v, i2_v, r1_v, r2_v, s1, s2):
        wid = lax.axis_index("s") * info.num_cores + lax.axis_index("c")
        base0 = wid * per_w

        def body(j, carry):
            base = base0 + j * chunk
            pltpu.sync_copy(i1_hbm.at[pl.ds(base, chunk)], i1_v)
            pltpu.sync_copy(i2_hbm.at[pl.ds(base, chunk)], i2_v)
            c1 = pltpu.async_copy(t_hbm.at[i1_v], r1_v, s1)
            c2 = pltpu.async_copy(t_hbm.at[i2_v], r2_v, s2)
            c1.wait()
            c2.wait()
            pltpu.sync_copy(r1_v, o1_hbm.at[pl.ds(base, chunk)])
            pltpu.sync_copy(r2_v, o2_hbm.at[pl.ds(base, chunk)])
            return carry

        lax.fori_loop(0, nch, body, 0)

    return gk(table, i1, i2)


def _proj_body(x_ref, wq_ref, bq_ref, wk_ref, bk_ref, wv_ref, bv_ref,
               ws_ref, bs_ref, t_ref):
    x = x_ref[...]
    q = jnp.dot(x, wq_ref[...], preferred_element_type=jnp.float32) + bq_ref[...]
    k = jnp.dot(x, wk_ref[...], preferred_element_type=jnp.float32) + bk_ref[...]
    v = jnp.dot(x, wv_ref[...], preferred_element_type=jnp.float32) + bv_ref[...]
    s = jnp.dot(x, ws_ref[...], preferred_element_type=jnp.float32) + bs_ref[...]
    t_ref[...] = jnp.concatenate([k, v, q, s], axis=1)


def _atom_proj(atoms, WqT, bq, WkT, bk, WvT, bv, WsT, bs):
    n = atoms.shape[0]
    blk = _pick_block(n, (5000, 4000, 2000, 1000, 800, 400, 200, 40, 8, 1))
    grid = (n // blk,)
    wspec = pl.BlockSpec((D, D), lambda i: (0, 0))
    bspec = pl.BlockSpec((D,), lambda i: (0,))
    return pl.pallas_call(
        _proj_body,
        grid=grid,
        in_specs=[pl.BlockSpec((blk, D), lambda i: (i, 0)),
                  wspec, bspec, wspec, bspec, wspec, bspec, wspec, bspec],
        out_specs=pl.BlockSpec((blk, 4 * D), lambda i: (i, 0)),
        out_shape=jax.ShapeDtypeStruct((n, 4 * D), jnp.float32),
    )(atoms, WqT, bq, WkT, bk, WvT, bv, WsT, bs)


def _edge_att_body(g1_ref, g2_ref, b_ref, we_ref, c_ref, ex_ref):
    e = jnp.dot(b_ref[...], we_ref[...], preferred_element_type=jnp.float32)
    g1 = g1_ref[...]
    kj = g1[:, :D] + e
    vj = g1[:, D:2 * D] + e
    lg = jnp.sum(g2_ref[:, 2 * D:3 * D] * kj, axis=1) * _INV_SQRT_D
    ex = jnp.exp(lg)
    c_ref[...] = (ex[:, None] * vj).astype(jnp.bfloat16)
    ex_ref[...] = ex[:, None]


def _edge_att(g1, g2, bonds, WeT):
    e = bonds.shape[0]
    blk = _pick_block(e, (6400, 3200, 1600, 800, 400, 200, 100, 50, 10, 1))
    grid = (e // blk,)
    return pl.pallas_call(
        _edge_att_body,
        grid=grid,
        in_specs=[pl.BlockSpec((blk, 4 * D), lambda i: (i, 0)),
                  pl.BlockSpec((blk, 4 * D), lambda i: (i, 0)),
                  pl.BlockSpec((blk, D), lambda i: (i, 0)),
                  pl.BlockSpec((D, D), lambda i: (0, 0))],
        out_specs=[pl.BlockSpec((blk, D), lambda i: (i, 0)),
                   pl.BlockSpec((blk, 1), lambda i: (i, 0))],
        out_shape=[jax.ShapeDtypeStruct((e, D), jnp.bfloat16),
                   jax.ShapeDtypeStruct((e, 1), jnp.float32)],
    )(g1, g2, bonds, WeT)


def _atom_upd_body(num_ref, den_ref, atoms_ref, t_ref,
                   a1_ref, ab1_ref, a2_ref, ab2_ref, out_ref, a3t_ref):
    den = jnp.sum(den_ref[...], axis=1)
    agg = num_ref[...].astype(jnp.float32) / (den[:, None] + 1e-16)
    atoms2 = atoms_ref[...] + agg + t_ref[:, 3 * D:]
    h = jnp.maximum(
        jnp.dot(atoms2, a1_ref[...], preferred_element_type=jnp.float32)
        + ab1_ref[...], 0.0)
    a3 = atoms2 + jnp.dot(h, a2_ref[...],
                          preferred_element_type=jnp.float32) + ab2_ref[...]
    out_ref[...] = a3
    a3t_ref[...] = jnp.concatenate(
        [a3, a3, jnp.zeros((a3.shape[0], 2 * D), jnp.float32)], axis=1)


def _atom_update(num, den2d, atoms, table, A1T, ab1, A2T, ab2):
    n = atoms.shape[0]
    p = den2d.shape[1]
    blk = 5000
    grid = (n // blk,)
    return pl.pallas_call(
        _atom_upd_body,
        grid=grid,
        in_specs=[pl.BlockSpec((blk, D), lambda i: (i, 0)),
                  pl.BlockSpec((blk, p), lambda i: (i, 0)),
                  pl.BlockSpec((blk, D), lambda i: (i, 0)),
                  pl.BlockSpec((blk, 4 * D), lambda i: (i, 0)),
                  pl.BlockSpec((D, 2 * D), lambda i: (0, 0)),
                  pl.BlockSpec((2 * D,), lambda i: (0,)),
                  pl.BlockSpec((2 * D, D), lambda i: (0, 0)),
                  pl.BlockSpec((D,), lambda i: (0,))],
        out_specs=[pl.BlockSpec((blk, D), lambda i: (i, 0)),
                   pl.BlockSpec((blk, 4 * D), lambda i: (i, 0))],
        out_shape=[jax.ShapeDtypeStruct((n, D), jnp.float32),
                   jax.ShapeDtypeStruct((n, 4 * D), jnp.float32)],
    )(num, den2d, atoms, table, A1T, ab1, A2T, ab2)


def _edge_mlp_body(ga1_ref, ga2_ref, b_ref, p1_ref, pb1_ref, p2_ref, pb2_ref,
                   p3_ref, pb3_ref, b1_ref, bb1_ref, b2_ref, bb2_ref, out_ref):
    bnd = b_ref[...]
    eb = jnp.concatenate([ga1_ref[:, :D], ga2_ref[:, :D], bnd], axis=1)
    h = jnp.maximum(
        jnp.dot(eb, p1_ref[...], preferred_element_type=jnp.float32) + pb1_ref[...], 0.0)
    h = jnp.maximum(
        jnp.dot(h, p2_ref[...], preferred_element_type=jnp.float32) + pb2_ref[...], 0.0)
    bonds2 = bnd + jnp.dot(h, p3_ref[...],
                           preferred_element_type=jnp.float32) + pb3_ref[...]
    h = jnp.maximum(
        jnp.dot(bonds2, b1_ref[...], preferred_element_type=jnp.float32) + bb1_ref[...],
        0.0)
    out_ref[...] = bonds2 + jnp.dot(h, b2_ref[...],
                                    preferred_element_type=jnp.float32) + bb2_ref[...]


def _edge_mlp(ga1, ga2, bonds, P1T, pb1, P2T, pb2, P3T, pb3, B1T, bb1, B2T, bb2):
    e = bonds.shape[0]
    blk = _pick_block(e, (6400, 3200, 1600, 800, 400, 200, 100, 50, 10, 1))
    grid = (e // blk,)
    def w(shape):
        return pl.BlockSpec(shape, lambda i: tuple(0 for _ in shape))
    return pl.pallas_call(
        _edge_mlp_body,
        grid=grid,
        in_specs=[pl.BlockSpec((blk, 4 * D), lambda i: (i, 0)),
                  pl.BlockSpec((blk, 4 * D), lambda i: (i, 0)),
                  pl.BlockSpec((blk, D), lambda i: (i, 0)),
                  w((3 * D, 4 * D)), w((4 * D,)),
                  w((4 * D, 2 * D)), w((2 * D,)),
                  w((2 * D, D)), w((D,)),
                  w((D, 2 * D)), w((2 * D,)),
                  w((2 * D, D)), w((D,))],
        out_specs=pl.BlockSpec((blk, D), lambda i: (i, 0)),
        out_shape=jax.ShapeDtypeStruct((e, D), jnp.float32),
    )(ga1, ga2, bonds, P1T, pb1, P2T, pb2, P3T, pb3, B1T, bb1, B2T, bb2)


def kernel(atoms, bonds, bond_atom_1, bond_atom_2,
           Wq, bq, Wk, bk, Wv, bv, We, Wskip, bskip,
           A1, ab1, A2, ab2,
           P1, pb1, P2, pb2, P3, pb3,
           B1, bb1, B2, bb2):
    src = bond_atom_1
    dst = bond_atom_2
    n = atoms.shape[0]

    table = _atom_proj(atoms, Wq.T, bq, Wk.T, bk, Wv.T, bv, Wskip.T, bskip)

    g1, g2 = _sc_gather2(table, src, dst)

    cnum, ex2d = _edge_att(g1, g2, bonds, We.T)

    num = jax.ops.segment_sum(cnum, dst, num_segments=n)
    den = jax.ops.segment_sum(ex2d[:, 0], dst, num_segments=n)

    atoms3, a3t = _atom_update(num, den[:, None], atoms, table,
                               A1.T, ab1, A2.T, ab2)

    ga1, ga2 = _sc_gather2(a3t, src, dst)

    bonds3 = _edge_mlp(ga1, ga2, bonds, P1.T, pb1, P2.T, pb2, P3.T, pb3,
                       B1.T, bb1, B2.T, bb2)
    return (atoms3, bonds3)


# speedup vs baseline: 4.1844x; 1.0172x over previous
import functools

import jax
import jax.numpy as jnp
from jax import lax
from jax.experimental import pallas as pl
from jax.experimental.pallas import tpu as pltpu
from jax.experimental.pallas import tpu_sc as plsc

D = 32
_INV_SQRT_D = 1.0 / (32.0 ** 0.5)


def _pick_block(n, candidates):
    for c in candidates:
        if n % c == 0 and (c % 8 == 0 or c == n):
            return c
    return n


def _sc_gather2(table, i1, i2, chunk=400):
    e = i1.shape[0]
    dt = table.shape[1]
    info = plsc.get_sparse_core_info()
    nw = info.num_cores * info.num_subcores
    per_w = e // nw
    nch = per_w // chunk
    mesh = plsc.VectorSubcoreMesh(core_axis_name="c", subcore_axis_name="s")

    @functools.partial(
        pl.kernel, mesh=mesh,
        out_type=[jax.ShapeDtypeStruct((e, dt), jnp.float32),
                  jax.ShapeDtypeStruct((e, dt), jnp.float32)],
        scratch_types=[pltpu.VMEM((chunk,), jnp.int32),
                       pltpu.VMEM((chunk,), jnp.int32),
                       pltpu.VMEM((chunk, dt), jnp.float32),
                       pltpu.VMEM((chunk, dt), jnp.float32),
                       pltpu.SemaphoreType.DMA,
                       pltpu.SemaphoreType.DMA,
                       pltpu.SemaphoreType.DMA,
                       pltpu.SemaphoreType.DMA])
    def gk(t_hbm, i1_hbm, i2_hbm, o1_hbm, o2_hbm,
           i1_v, i2_v, r1_v, r2_v, s1, s2, s3, s4):
        wid = lax.axis_index("s") * info.num_cores + lax.axis_index("c")
        base0 = wid * per_w

        def body(j, carry):
            base = base0 + j * chunk
            pltpu.sync_copy(i1_hbm.at[pl.ds(base, chunk)], i1_v)
            pltpu.sync_copy(i2_hbm.at[pl.ds(base, chunk)], i2_v)

            @pl.when(j > 0)
            def _():
                pb = base - chunk
                pltpu.make_async_copy(
                    r1_v, o1_hbm.at[pl.ds(pb, chunk)], s3).wait()
                pltpu.make_async_copy(
                    r2_v, o2_hbm.at[pl.ds(pb, chunk)], s4).wait()

            c1 = pltpu.async_copy(t_hbm.at[i1_v], r1_v, s1)
            c2 = pltpu.async_copy(t_hbm.at[i2_v], r2_v, s2)
            c1.wait()
            c2.wait()
            pltpu.async_copy(r1_v, o1_hbm.at[pl.ds(base, chunk)], s3)
            pltpu.async_copy(r2_v, o2_hbm.at[pl.ds(base, chunk)], s4)
            return carry

        lax.fori_loop(0, nch, body, 0)
        last = base0 + (nch - 1) * chunk
        pltpu.make_async_copy(r1_v, o1_hbm.at[pl.ds(last, chunk)], s3).wait()
        pltpu.make_async_copy(r2_v, o2_hbm.at[pl.ds(last, chunk)], s4).wait()

    return gk(table, i1, i2)


def _proj_body(x_ref, wq_ref, bq_ref, wk_ref, bk_ref, wv_ref, bv_ref,
               ws_ref, bs_ref, t_ref):
    x = x_ref[...]
    q = jnp.dot(x, wq_ref[...], preferred_element_type=jnp.float32) + bq_ref[...]
    k = jnp.dot(x, wk_ref[...], preferred_element_type=jnp.float32) + bk_ref[...]
    v = jnp.dot(x, wv_ref[...], preferred_element_type=jnp.float32) + bv_ref[...]
    s = jnp.dot(x, ws_ref[...], preferred_element_type=jnp.float32) + bs_ref[...]
    t_ref[...] = jnp.concatenate([k, v, q, s], axis=1)


def _atom_proj(atoms, WqT, bq, WkT, bk, WvT, bv, WsT, bs):
    n = atoms.shape[0]
    blk = _pick_block(n, (5000, 4000, 2000, 1000, 800, 400, 200, 40, 8, 1))
    grid = (n // blk,)
    wspec = pl.BlockSpec((D, D), lambda i: (0, 0))
    bspec = pl.BlockSpec((D,), lambda i: (0,))
    return pl.pallas_call(
        _proj_body,
        grid=grid,
        in_specs=[pl.BlockSpec((blk, D), lambda i: (i, 0)),
                  wspec, bspec, wspec, bspec, wspec, bspec, wspec, bspec],
        out_specs=pl.BlockSpec((blk, 4 * D), lambda i: (i, 0)),
        out_shape=jax.ShapeDtypeStruct((n, 4 * D), jnp.float32),
    )(atoms, WqT, bq, WkT, bk, WvT, bv, WsT, bs)


def _edge_att_body(g1_ref, g2_ref, b_ref, we_ref, c_ref, ex_ref):
    e = jnp.dot(b_ref[...], we_ref[...], preferred_element_type=jnp.float32)
    g1 = g1_ref[...]
    kj = g1[:, :D] + e
    vj = g1[:, D:2 * D] + e
    lg = jnp.sum(g2_ref[:, 2 * D:3 * D] * kj, axis=1) * _INV_SQRT_D
    ex = jnp.exp(lg)
    c_ref[...] = (ex[:, None] * vj).astype(jnp.bfloat16)
    ex_ref[...] = ex[:, None]


def _edge_att(g1, g2, bonds, WeT):
    e = bonds.shape[0]
    blk = _pick_block(e, (6400, 3200, 1600, 800, 400, 200, 100, 50, 10, 1))
    grid = (e // blk,)
    return pl.pallas_call(
        _edge_att_body,
        grid=grid,
        in_specs=[pl.BlockSpec((blk, 4 * D), lambda i: (i, 0)),
                  pl.BlockSpec((blk, 4 * D), lambda i: (i, 0)),
                  pl.BlockSpec((blk, D), lambda i: (i, 0)),
                  pl.BlockSpec((D, D), lambda i: (0, 0))],
        out_specs=[pl.BlockSpec((blk, D), lambda i: (i, 0)),
                   pl.BlockSpec((blk, 1), lambda i: (i, 0))],
        out_shape=[jax.ShapeDtypeStruct((e, D), jnp.bfloat16),
                   jax.ShapeDtypeStruct((e, 1), jnp.float32)],
    )(g1, g2, bonds, WeT)


def _atom_upd_body(num_ref, den_ref, atoms_ref, t_ref,
                   a1_ref, ab1_ref, a2_ref, ab2_ref, out_ref, a3t_ref):
    den = jnp.sum(den_ref[...], axis=1)
    agg = num_ref[...].astype(jnp.float32) / (den[:, None] + 1e-16)
    atoms2 = atoms_ref[...] + agg + t_ref[:, 3 * D:]
    h = jnp.maximum(
        jnp.dot(atoms2, a1_ref[...], preferred_element_type=jnp.float32)
        + ab1_ref[...], 0.0)
    a3 = atoms2 + jnp.dot(h, a2_ref[...],
                          preferred_element_type=jnp.float32) + ab2_ref[...]
    out_ref[...] = a3
    a3t_ref[...] = jnp.concatenate(
        [a3, a3, jnp.zeros((a3.shape[0], 2 * D), jnp.float32)], axis=1)


def _atom_update(num, den2d, atoms, table, A1T, ab1, A2T, ab2):
    n = atoms.shape[0]
    p = den2d.shape[1]
    blk = 5000
    grid = (n // blk,)
    return pl.pallas_call(
        _atom_upd_body,
        grid=grid,
        in_specs=[pl.BlockSpec((blk, D), lambda i: (i, 0)),
                  pl.BlockSpec((blk, p), lambda i: (i, 0)),
                  pl.BlockSpec((blk, D), lambda i: (i, 0)),
                  pl.BlockSpec((blk, 4 * D), lambda i: (i, 0)),
                  pl.BlockSpec((D, 2 * D), lambda i: (0, 0)),
                  pl.BlockSpec((2 * D,), lambda i: (0,)),
                  pl.BlockSpec((2 * D, D), lambda i: (0, 0)),
                  pl.BlockSpec((D,), lambda i: (0,))],
        out_specs=[pl.BlockSpec((blk, D), lambda i: (i, 0)),
                   pl.BlockSpec((blk, 4 * D), lambda i: (i, 0))],
        out_shape=[jax.ShapeDtypeStruct((n, D), jnp.float32),
                   jax.ShapeDtypeStruct((n, 4 * D), jnp.float32)],
    )(num, den2d, atoms, table, A1T, ab1, A2T, ab2)


def _edge_mlp_body(ga1_ref, ga2_ref, b_ref, p1_ref, pb1_ref, p2_ref, pb2_ref,
                   p3_ref, pb3_ref, b1_ref, bb1_ref, b2_ref, bb2_ref, out_ref):
    bnd = b_ref[...]
    eb = jnp.concatenate([ga1_ref[:, :D], ga2_ref[:, :D], bnd], axis=1)
    h = jnp.maximum(
        jnp.dot(eb, p1_ref[...], preferred_element_type=jnp.float32) + pb1_ref[...], 0.0)
    h = jnp.maximum(
        jnp.dot(h, p2_ref[...], preferred_element_type=jnp.float32) + pb2_ref[...], 0.0)
    bonds2 = bnd + jnp.dot(h, p3_ref[...],
                           preferred_element_type=jnp.float32) + pb3_ref[...]
    h = jnp.maximum(
        jnp.dot(bonds2, b1_ref[...], preferred_element_type=jnp.float32) + bb1_ref[...],
        0.0)
    out_ref[...] = bonds2 + jnp.dot(h, b2_ref[...],
                                    preferred_element_type=jnp.float32) + bb2_ref[...]


def _edge_mlp(ga1, ga2, bonds, P1T, pb1, P2T, pb2, P3T, pb3, B1T, bb1, B2T, bb2):
    e = bonds.shape[0]
    blk = _pick_block(e, (6400, 3200, 1600, 800, 400, 200, 100, 50, 10, 1))
    grid = (e // blk,)
    def w(shape):
        return pl.BlockSpec(shape, lambda i: tuple(0 for _ in shape))
    return pl.pallas_call(
        _edge_mlp_body,
        grid=grid,
        in_specs=[pl.BlockSpec((blk, 4 * D), lambda i: (i, 0)),
                  pl.BlockSpec((blk, 4 * D), lambda i: (i, 0)),
                  pl.BlockSpec((blk, D), lambda i: (i, 0)),
                  w((3 * D, 4 * D)), w((4 * D,)),
                  w((4 * D, 2 * D)), w((2 * D,)),
                  w((2 * D, D)), w((D,)),
                  w((D, 2 * D)), w((2 * D,)),
                  w((2 * D, D)), w((D,))],
        out_specs=pl.BlockSpec((blk, D), lambda i: (i, 0)),
        out_shape=jax.ShapeDtypeStruct((e, D), jnp.float32),
    )(ga1, ga2, bonds, P1T, pb1, P2T, pb2, P3T, pb3, B1T, bb1, B2T, bb2)


def kernel(atoms, bonds, bond_atom_1, bond_atom_2,
           Wq, bq, Wk, bk, Wv, bv, We, Wskip, bskip,
           A1, ab1, A2, ab2,
           P1, pb1, P2, pb2, P3, pb3,
           B1, bb1, B2, bb2):
    src = bond_atom_1
    dst = bond_atom_2
    n = atoms.shape[0]

    table = _atom_proj(atoms, Wq.T, bq, Wk.T, bk, Wv.T, bv, Wskip.T, bskip)

    g1, g2 = _sc_gather2(table, src, dst)

    cnum, ex2d = _edge_att(g1, g2, bonds, We.T)

    num = jax.ops.segment_sum(cnum, dst, num_segments=n)
    den = jax.ops.segment_sum(ex2d[:, 0], dst, num_segments=n)

    atoms3, a3t = _atom_update(num, den[:, None], atoms, table,
                               A1.T, ab1, A2.T, ab2)

    ga1, ga2 = _sc_gather2(a3t, src, dst)

    bonds3 = _edge_mlp(ga1, ga2, bonds, P1.T, pb1, P2.T, pb2, P3.T, pb3,
                       B1.T, bb1, B2.T, bb2)
    return (atoms3, bonds3)
